# baseline (speedup 1.0000x reference)
import functools

import jax
import jax.numpy as jnp
from jax import lax
from jax.experimental import pallas as pl
from jax.experimental.pallas import tpu as pltpu
from jax.experimental.pallas import tpu_sc as plsc

N_USERS = 100000
N_ITEMS = 1000000
HIDDEN = 64
BATCH = 16384

_NC = 2
_NS = 16
_NW = _NC * _NS
_BPW = BATCH // _NW
_L = 16


def _bpr_body(users_hbm, items_hbm, ua_hbm, ia_hbm, ub_hbm, ib_hbm, out_hbm,
              uidx_v, iidx_v, ua_v, ia_v, ub_v, ib_v, out_v,
              sem0, sem1, sem2, sem3):
    wid = lax.axis_index("s") * _NC + lax.axis_index("c")
    base = wid * _BPW

    pltpu.sync_copy(users_hbm.at[pl.ds(base, _BPW)], uidx_v)
    pltpu.sync_copy(items_hbm.at[pl.ds(base, _BPW)], iidx_v)

    c0 = pltpu.async_copy(ua_hbm.at[uidx_v], ua_v, sem0)
    c1 = pltpu.async_copy(ia_hbm.at[iidx_v], ia_v, sem1)
    c2 = pltpu.async_copy(ub_hbm.at[uidx_v], ub_v, sem2)
    c3 = pltpu.async_copy(ib_hbm.at[iidx_v], ib_v, sem3)
    c0.wait()
    c1.wait()
    c2.wait()
    c3.wait()

    def group(g, carry):
        r0 = g * _L
        rows = lax.iota(jnp.int32, _L) + r0
        acc = ub_v[pl.ds(r0, _L)] + ib_v[pl.ds(r0, _L)]

        def hstep(h, a):
            cols = jnp.full((_L,), h, jnp.int32)
            return a + (plsc.load_gather(ua_v, [rows, cols])
                        * plsc.load_gather(ia_v, [rows, cols]))

        acc = lax.fori_loop(0, HIDDEN, hstep, acc)
        out_v[pl.ds(r0, _L)] = acc
        return carry

    lax.fori_loop(0, _BPW // _L, group, 0)
    pltpu.sync_copy(out_v, out_hbm.at[pl.ds(base, _BPW)])


@jax.jit
def _bpr(users, items, user_alpha, item_alpha, user_beta, item_beta):
    mesh = plsc.VectorSubcoreMesh(core_axis_name="c", subcore_axis_name="s")
    run = functools.partial(
        pl.kernel,
        mesh=mesh,
        compiler_params=pltpu.CompilerParams(
            needs_layout_passes=False, use_tc_tiling_on_sc=False),
        out_type=jax.ShapeDtypeStruct((BATCH,), jnp.float32),
        scratch_types=[
            pltpu.VMEM((_BPW,), jnp.int32),
            pltpu.VMEM((_BPW,), jnp.int32),
            pltpu.VMEM((_BPW, HIDDEN), jnp.float32),
            pltpu.VMEM((_BPW, HIDDEN), jnp.float32),
            pltpu.VMEM((_BPW,), jnp.float32),
            pltpu.VMEM((_BPW,), jnp.float32),
            pltpu.VMEM((_BPW,), jnp.float32),
            pltpu.SemaphoreType.DMA,
            pltpu.SemaphoreType.DMA,
            pltpu.SemaphoreType.DMA,
            pltpu.SemaphoreType.DMA,
        ],
    )(_bpr_body)
    return run(users, items, user_alpha, item_alpha, user_beta, item_beta)


def kernel(users, items, user_alpha, item_alpha, user_beta, item_beta):
    users = users.astype(jnp.int32)
    items = items.astype(jnp.int32)
    user_beta = user_beta.reshape(-1)
    item_beta = item_beta.reshape(-1)
    return _bpr(users, items, user_alpha, item_alpha, user_beta, item_beta)
